# baseline (speedup 1.0000x reference)
import jax
import jax.numpy as jnp
from jax import lax
from jax.experimental import pallas as pl
from jax.experimental.pallas import tpu as pltpu
from jax.experimental.pallas import tpu_sc as plsc

N = 100000
NREL = 17
B = 10000
S = 10
D = 128
L = B * S

NC = 2
NS = 16
NW = NC * NS
DPW = D // NW
TROWS = 782
LP = 102400
IR = LP // 128
BLK = 32
NBLK = IR // BLK
GFULL = 4096 // 16
GTAIL = (L - (NBLK - 1) * 4096) // 16


def _p1_body(tab_hbm, pk_hbm, val_hbm, table_v, idx_v, out_v):
    wid = lax.axis_index("s") * NC + lax.axis_index("c")
    m17 = jnp.full((16,), 0x1FFFF, jnp.int32)
    m7 = jnp.full((16,), 0x7F, jnp.int32)
    rhi = jnp.full((16,), TROWS - 1, jnp.int32)
    roff = jnp.full((16,), 32, jnp.int32)

    for k in range(DPW):
        dd = wid * DPW + k
        pltpu.sync_copy(tab_hbm.at[dd], table_v)

        def blk_fn(c, carry):
            pltpu.sync_copy(pk_hbm.at[pl.ds(c * BLK, BLK)], idx_v)

            def grp_fn(g, carry2):
                r, col = g // 8, (g % 8) * 16
                sl = pl.ds(col, 16)
                pk = idx_v[r, sl]
                n = lax.bitwise_and(pk, m17)
                vn = plsc.load_gather(
                    table_v,
                    [lax.shift_right_logical(n, 7), lax.bitwise_and(n, m7)],
                )
                vr = plsc.load_gather(
                    table_v,
                    [rhi, lax.shift_right_logical(pk, 17) + roff],
                )
                out_v[r, sl] = vn + vr
                return carry2

            ngrp = jnp.where(c == NBLK - 1, GTAIL, GFULL)
            lax.fori_loop(0, ngrp, grp_fn, 0)
            pltpu.sync_copy(out_v, val_hbm.at[dd, pl.ds(c * BLK, BLK)])
            return carry

        lax.fori_loop(0, NBLK, blk_fn, 0)


def _p2_body(val_ref, out_ref):
    v = val_ref[...]
    out_ref[...] = jnp.maximum(jnp.sum(v, axis=1) * jnp.float32(1.0 / S), 0.0)


@jax.jit
def kernel(sampled_neighbors, sampled_relations, weight, relation_weight):
    nf = sampled_neighbors.reshape(-1).astype(jnp.int32)
    rf = sampled_relations.reshape(-1).astype(jnp.int32)
    pk = nf | (rf << 17)
    pk = jnp.pad(pk, (0, LP - L)).reshape(IR, 128)
    tab = jnp.concatenate([weight, relation_weight], axis=1)
    tab = jnp.pad(tab, ((0, 0), (0, TROWS * 128 - (N + NREL))))
    tab = tab.reshape(D, TROWS, 128)

    mesh = plsc.VectorSubcoreMesh(
        core_axis_name="c", subcore_axis_name="s", num_cores=NC, num_subcores=NS
    )
    val = pl.kernel(
        _p1_body,
        out_type=jax.ShapeDtypeStruct((D, IR, 128), jnp.float32),
        mesh=mesh,
        scratch_types=[
            pltpu.VMEM((TROWS, 128), jnp.float32),
            pltpu.VMEM((BLK, 128), jnp.int32),
            pltpu.VMEM((BLK, 128), jnp.float32),
        ],
        compiler_params=pltpu.CompilerParams(needs_layout_passes=False),
    )(tab, pk)

    val_flat = val.reshape(D, LP)[:, :L].reshape(B, S, D)
    bs = 2000
    out = pl.pallas_call(
        _p2_body,
        out_shape=jax.ShapeDtypeStruct((B, D), jnp.float32),
        grid=(B // bs,),
        in_specs=[pl.BlockSpec((bs, S, D), lambda i: (i, 0, 0))],
        out_specs=pl.BlockSpec((bs, D), lambda i: (i, 0)),
    )(val_flat)
    return out


# speedup vs baseline: 1.4579x; 1.4579x over previous
import jax
import jax.numpy as jnp
from jax import lax
from jax.experimental import pallas as pl
from jax.experimental.pallas import tpu as pltpu
from jax.experimental.pallas import tpu_sc as plsc

N = 100000
NREL = 17
B = 10000
S = 10
D = 128
L = B * S

NC = 2
NS = 16
NW = NC * NS
DPW = D // NW
TROWS = 782
BLK = 32
NBLK = 26
LP = NBLK * 4096
IR = LP // 128


def _p1_body(tab_hbm, pk_hbm, val_hbm, table_v, i0, i1, o0, o1,
             si0, si1, so0, so1):
    wid = lax.axis_index("s") * NC + lax.axis_index("c")
    m17 = jnp.full((16,), 0x1FFFF, jnp.int32)
    m7 = jnp.full((16,), 0x7F, jnp.int32)
    rhi = jnp.full((16,), TROWS - 1, jnp.int32)
    roff = jnp.full((16,), 32, jnp.int32)

    bufs = ((i0, o0, si0, so0), (i1, o1, si1, so1))

    def idx_src(c):
        return pk_hbm.at[pl.ds(c * BLK, BLK)]

    for k in range(DPW):
        dd = wid * DPW + k
        pltpu.sync_copy(tab_hbm.at[dd], table_v)
        pltpu.async_copy(idx_src(0), i0, si0)
        pltpu.async_copy(idx_src(1), i1, si1)

        def super_fn(h, carry):
            for par, (ibuf, obuf, sin, sout) in enumerate(bufs):
                c = 2 * h + par
                pltpu.make_async_copy(idx_src(c), ibuf, sin).wait()

                @pl.when(h > 0)
                def _():
                    pltpu.make_async_copy(
                        obuf, val_hbm.at[dd, pl.ds((c - 2) * BLK, BLK)], sout
                    ).wait()

                def row_fn(r, carry2):
                    for g in range(8):
                        sl = pl.ds(g * 16, 16)
                        pk = ibuf[r, sl]
                        n = lax.bitwise_and(pk, m17)
                        vn = plsc.load_gather(
                            table_v,
                            [lax.shift_right_logical(n, 7),
                             lax.bitwise_and(n, m7)],
                        )
                        vr = plsc.load_gather(
                            table_v,
                            [rhi, lax.shift_right_logical(pk, 17) + roff],
                        )
                        obuf[r, sl] = vn + vr
                    return carry2

                lax.fori_loop(0, BLK, row_fn, 0)
                pltpu.async_copy(
                    obuf, val_hbm.at[dd, pl.ds(c * BLK, BLK)], sout
                )

                @pl.when(h < NBLK // 2 - 1)
                def _():
                    pltpu.async_copy(idx_src(c + 2), ibuf, sin)

            return carry

        lax.fori_loop(0, NBLK // 2, super_fn, 0)
        pltpu.make_async_copy(
            o0, val_hbm.at[dd, pl.ds((NBLK - 2) * BLK, BLK)], so0
        ).wait()
        pltpu.make_async_copy(
            o1, val_hbm.at[dd, pl.ds((NBLK - 1) * BLK, BLK)], so1
        ).wait()


def _p2_body(val_ref, out_ref):
    v = val_ref[...]
    out_ref[...] = jnp.maximum(jnp.sum(v, axis=1) * jnp.float32(1.0 / S), 0.0)


@jax.jit
def kernel(sampled_neighbors, sampled_relations, weight, relation_weight):
    nf = sampled_neighbors.reshape(-1).astype(jnp.int32)
    rf = sampled_relations.reshape(-1).astype(jnp.int32)
    pk = nf | (rf << 17)
    pk = jnp.pad(pk, (0, LP - L)).reshape(IR, 128)
    tab = jnp.concatenate([weight, relation_weight], axis=1)
    tab = jnp.pad(tab, ((0, 0), (0, TROWS * 128 - (N + NREL))))
    tab = tab.reshape(D, TROWS, 128)

    mesh = plsc.VectorSubcoreMesh(
        core_axis_name="c", subcore_axis_name="s", num_cores=NC, num_subcores=NS
    )
    val = pl.kernel(
        _p1_body,
        out_type=jax.ShapeDtypeStruct((D, IR, 128), jnp.float32),
        mesh=mesh,
        scratch_types=[
            pltpu.VMEM((TROWS, 128), jnp.float32),
            pltpu.VMEM((BLK, 128), jnp.int32),
            pltpu.VMEM((BLK, 128), jnp.int32),
            pltpu.VMEM((BLK, 128), jnp.float32),
            pltpu.VMEM((BLK, 128), jnp.float32),
            pltpu.SemaphoreType.DMA,
            pltpu.SemaphoreType.DMA,
            pltpu.SemaphoreType.DMA,
            pltpu.SemaphoreType.DMA,
        ],
        compiler_params=pltpu.CompilerParams(needs_layout_passes=False),
    )(tab, pk)

    val_flat = val.reshape(D, LP)[:, :L].reshape(B, S, D)
    bs = 2000
    out = pl.pallas_call(
        _p2_body,
        out_shape=jax.ShapeDtypeStruct((B, D), jnp.float32),
        grid=(B // bs,),
        in_specs=[pl.BlockSpec((bs, S, D), lambda i: (i, 0, 0))],
        out_specs=pl.BlockSpec((bs, D), lambda i: (i, 0)),
    )(val_flat)
    return out


# speedup vs baseline: 1.5414x; 1.0573x over previous
import jax
import jax.numpy as jnp
from jax import lax
from jax.experimental import pallas as pl
from jax.experimental.pallas import tpu as pltpu
from jax.experimental.pallas import tpu_sc as plsc

N = 100000
NREL = 17
B = 10000
S = 10
D = 128
L = B * S

NC = 2
NS = 16
NW = NC * NS
DPW = D // NW
TROWS = 782
BLK = 32
NBLK = 26
LP = NBLK * 4096
IR = LP // 128


def _p1_body(w_hbm, side_hbm, pk_hbm, val_hbm, table_v, i0, i1, o0, o1,
             si0, si1, so0, so1):
    wid = lax.axis_index("s") * NC + lax.axis_index("c")
    m17 = jnp.full((16,), 0x1FFFF, jnp.int32)
    roff = jnp.full((16,), N, jnp.int32)

    bufs = ((i0, o0, si0, so0), (i1, o1, si1, so1))

    NAL = (N // 128) * 128

    def idx_src(c):
        return pk_hbm.at[pl.ds(c * BLK, BLK)]

    for k in range(DPW):
        dd = wid * DPW + k
        pltpu.sync_copy(w_hbm.at[dd, pl.ds(0, NAL)], table_v.at[pl.ds(0, NAL)])
        pltpu.sync_copy(side_hbm.at[dd], table_v.at[pl.ds(NAL, 256)])
        pltpu.async_copy(idx_src(0), i0, si0)
        pltpu.async_copy(idx_src(1), i1, si1)

        def super_fn(h, carry):
            for par, (ibuf, obuf, sin, sout) in enumerate(bufs):
                c = 2 * h + par
                pltpu.make_async_copy(idx_src(c), ibuf, sin).wait()

                @pl.when(h > 0)
                def _():
                    pltpu.make_async_copy(
                        obuf, val_hbm.at[dd, pl.ds((c - 2) * BLK, BLK)], sout
                    ).wait()

                def row_fn(r, carry2):
                    for g in range(8):
                        sl = pl.ds(g * 16, 16)
                        pk = ibuf[r, sl]
                        n = lax.bitwise_and(pk, m17)
                        vn = plsc.load_gather(table_v, [n])
                        vr = plsc.load_gather(
                            table_v, [lax.shift_right_logical(pk, 17) + roff]
                        )
                        obuf[r, sl] = vn + vr
                    return carry2

                lax.fori_loop(0, BLK, row_fn, 0)
                pltpu.async_copy(
                    obuf, val_hbm.at[dd, pl.ds(c * BLK, BLK)], sout
                )

                @pl.when(h < NBLK // 2 - 1)
                def _():
                    pltpu.async_copy(idx_src(c + 2), ibuf, sin)

            return carry

        lax.fori_loop(0, NBLK // 2, super_fn, 0)
        pltpu.make_async_copy(
            o0, val_hbm.at[dd, pl.ds((NBLK - 2) * BLK, BLK)], so0
        ).wait()
        pltpu.make_async_copy(
            o1, val_hbm.at[dd, pl.ds((NBLK - 1) * BLK, BLK)], so1
        ).wait()


def _p2_body(val_ref, out_ref):
    v = val_ref[...]
    out_ref[...] = jnp.maximum(jnp.sum(v, axis=1) * jnp.float32(1.0 / S), 0.0)


@jax.jit
def kernel(sampled_neighbors, sampled_relations, weight, relation_weight):
    nf = sampled_neighbors.reshape(-1).astype(jnp.int32)
    rf = sampled_relations.reshape(-1).astype(jnp.int32)
    pk = nf | (rf << 17)
    pk = jnp.pad(pk, (0, LP - L)).reshape(IR, 128)
    side = jnp.concatenate(
        [weight[:, (N // 128) * 128:], relation_weight,
         jnp.zeros((D, 256 - 32 - NREL), jnp.float32)], axis=1)

    mesh = plsc.VectorSubcoreMesh(
        core_axis_name="c", subcore_axis_name="s", num_cores=NC, num_subcores=NS
    )
    val = pl.kernel(
        _p1_body,
        out_type=jax.ShapeDtypeStruct((D, IR, 128), jnp.float32),
        mesh=mesh,
        scratch_types=[
            pltpu.VMEM((N + 256 - 32,), jnp.float32),
            pltpu.VMEM((BLK, 128), jnp.int32),
            pltpu.VMEM((BLK, 128), jnp.int32),
            pltpu.VMEM((BLK, 128), jnp.float32),
            pltpu.VMEM((BLK, 128), jnp.float32),
            pltpu.SemaphoreType.DMA,
            pltpu.SemaphoreType.DMA,
            pltpu.SemaphoreType.DMA,
            pltpu.SemaphoreType.DMA,
        ],
        compiler_params=pltpu.CompilerParams(needs_layout_passes=False),
    )(weight, side, pk)

    val_flat = val.reshape(D, LP)[:, :L].reshape(B, S, D)
    bs = 2000
    out = pl.pallas_call(
        _p2_body,
        out_shape=jax.ShapeDtypeStruct((B, D), jnp.float32),
        grid=(B // bs,),
        in_specs=[pl.BlockSpec((bs, S, D), lambda i: (i, 0, 0))],
        out_specs=pl.BlockSpec((bs, D), lambda i: (i, 0)),
    )(val_flat)
    return out


# speedup vs baseline: 1.6061x; 1.0419x over previous
import jax
import jax.numpy as jnp
from jax import lax
from jax.experimental import pallas as pl
from jax.experimental.pallas import tpu as pltpu
from jax.experimental.pallas import tpu_sc as plsc

N = 100000
NREL = 17
B = 10000
S = 10
D = 128
L = B * S

NC = 2
NS = 16
NW = NC * NS
DPW = D // NW
TROWS = 782
BLK = 32
NBLK = 26
LP = NBLK * 4096
IR = LP // 128


def _p1_body(w_hbm, side_hbm, pk_hbm, val_hbm, table_v, i0, i1, o0, o1,
             si0, si1, so0, so1):
    wid = lax.axis_index("s") * NC + lax.axis_index("c")
    m17 = jnp.full((16,), 0x1FFFF, jnp.int32)
    roff = jnp.full((16,), N, jnp.int32)

    bufs = ((i0, o0, si0, so0), (i1, o1, si1, so1))

    NAL = (N // 128) * 128

    def idx_src(c):
        return pk_hbm.at[pl.ds(c * BLK, BLK)]

    for k in range(DPW):
        dd = wid * DPW + k
        pltpu.sync_copy(w_hbm.at[dd, pl.ds(0, NAL)], table_v.at[pl.ds(0, NAL)])
        pltpu.sync_copy(side_hbm.at[dd], table_v.at[pl.ds(NAL, 256)])
        pltpu.async_copy(idx_src(0), i0, si0)
        pltpu.async_copy(idx_src(1), i1, si1)

        def super_fn(h, carry):
            for par, (ibuf, obuf, sin, sout) in enumerate(bufs):
                c = 2 * h + par
                pltpu.make_async_copy(idx_src(c), ibuf, sin).wait()

                @pl.when(h > 0)
                def _():
                    pltpu.make_async_copy(
                        obuf, val_hbm.at[dd, pl.ds((c - 2) * BLK, BLK)], sout
                    ).wait()

                def row_fn(r, carry2):
                    for g in range(8):
                        sl = pl.ds(g * 16, 16)
                        pk = ibuf[r, sl]
                        n = lax.bitwise_and(pk, m17)
                        vn = plsc.load_gather(table_v, [n])
                        vr = plsc.load_gather(
                            table_v, [lax.shift_right_logical(pk, 17) + roff]
                        )
                        obuf[r, sl] = vn + vr
                    return carry2

                lax.fori_loop(0, BLK, row_fn, 0)
                pltpu.async_copy(
                    obuf, val_hbm.at[dd, pl.ds(c * BLK, BLK)], sout
                )

                @pl.when(h < NBLK // 2 - 1)
                def _():
                    pltpu.async_copy(idx_src(c + 2), ibuf, sin)

            return carry

        lax.fori_loop(0, NBLK // 2, super_fn, 0)
        pltpu.make_async_copy(
            o0, val_hbm.at[dd, pl.ds((NBLK - 2) * BLK, BLK)], so0
        ).wait()
        pltpu.make_async_copy(
            o1, val_hbm.at[dd, pl.ds((NBLK - 1) * BLK, BLK)], so1
        ).wait()


def _p2_body(val_ref, out_ref):
    v = val_ref[...]
    bs = v.shape[0] // S
    r = v.reshape(bs, S, D)
    out_ref[...] = jnp.maximum(jnp.sum(r, axis=1) * jnp.float32(1.0 / S), 0.0)


@jax.jit
def kernel(sampled_neighbors, sampled_relations, weight, relation_weight):
    nf = sampled_neighbors.reshape(-1).astype(jnp.int32)
    rf = sampled_relations.reshape(-1).astype(jnp.int32)
    pk = nf | (rf << 17)
    pk = jnp.pad(pk, (0, LP - L)).reshape(IR, 128)
    side = jnp.concatenate(
        [weight[:, (N // 128) * 128:], relation_weight,
         jnp.zeros((D, 256 - 32 - NREL), jnp.float32)], axis=1)

    mesh = plsc.VectorSubcoreMesh(
        core_axis_name="c", subcore_axis_name="s", num_cores=NC, num_subcores=NS
    )
    val = pl.kernel(
        _p1_body,
        out_type=jax.ShapeDtypeStruct((D, IR, 128), jnp.float32),
        mesh=mesh,
        scratch_types=[
            pltpu.VMEM((N + 256 - 32,), jnp.float32),
            pltpu.VMEM((BLK, 128), jnp.int32),
            pltpu.VMEM((BLK, 128), jnp.int32),
            pltpu.VMEM((BLK, 128), jnp.float32),
            pltpu.VMEM((BLK, 128), jnp.float32),
            pltpu.SemaphoreType.DMA,
            pltpu.SemaphoreType.DMA,
            pltpu.SemaphoreType.DMA,
            pltpu.SemaphoreType.DMA,
        ],
        compiler_params=pltpu.CompilerParams(needs_layout_passes=False),
    )(weight, side, pk)

    val_rows = val.reshape(D, LP)[:, :L].reshape(B * S, D)
    bs = 2000
    out = pl.pallas_call(
        _p2_body,
        out_shape=jax.ShapeDtypeStruct((B, D), jnp.float32),
        grid=(B // bs,),
        in_specs=[pl.BlockSpec((bs * S, D), lambda i: (i, 0))],
        out_specs=pl.BlockSpec((bs, D), lambda i: (i, 0)),
    )(val_rows)
    return out


# speedup vs baseline: 1.6561x; 1.0311x over previous
import jax
import jax.numpy as jnp
from jax import lax
from jax.experimental import pallas as pl
from jax.experimental.pallas import tpu as pltpu
from jax.experimental.pallas import tpu_sc as plsc

N = 100000
NREL = 17
B = 10000
S = 10
D = 128
L = B * S

NC = 2
NS = 16
NW = NC * NS
DPW = D // NW
WPT = DPW * L
RPT = WPT // 128
BLK = 32
NFULL = WPT // 4096
TAILG = (WPT - NFULL * 4096) // 16
TAILW = WPT - NFULL * 4096
NBLK = NFULL + 1
IRP = NBLK * BLK
NAL = (N // 128) * 128
SPLITS = {1: (24, 106), 2: (48, 212), 3: (73, 62)}


def _p1_body(w_hbm, side_hbm, pk_hbm, val_hbm, table_v, i0, i1, o0, o1,
             si0, si1, so0, so1):
    wid = lax.axis_index("s") * NC + lax.axis_index("c")
    m17 = jnp.full((16,), 0x1FFFF, jnp.int32)
    roff = jnp.full((16,), N, jnp.int32)
    wbase = wid * WPT

    def stage_table(k):
        dd = wid * DPW + k
        pltpu.sync_copy(w_hbm.at[dd, pl.ds(0, NAL)], table_v.at[pl.ds(0, NAL)])
        pltpu.sync_copy(side_hbm.at[dd], table_v.at[pl.ds(NAL, 256)])

    def idx_src(c):
        return pk_hbm.at[pl.ds(c * 4096, 4096)]

    stage_table(0)
    pltpu.async_copy(idx_src(0), i0, si0)
    pltpu.async_copy(idx_src(1), i1, si1)

    bufs = ((i0, o0, si0, so0), (i1, o1, si1, so1))

    def super_fn(h, carry):
        for par, (ibuf, obuf, sin, sout) in enumerate(bufs):
            c = 2 * h + par
            pltpu.make_async_copy(idx_src(c), ibuf, sin).wait()

            @pl.when(h > 0)
            def _():
                pltpu.make_async_copy(
                    obuf, val_hbm.at[pl.ds(wbase + (c - 2) * 4096, 4096)], sout
                ).wait()

            def row_fn(g, carry2):
                sl = pl.ds(g * 16, 16)
                pk = ibuf[sl]
                n = lax.bitwise_and(pk, m17)
                vn = plsc.load_gather(table_v, [n])
                vr = plsc.load_gather(
                    table_v, [lax.shift_right_logical(pk, 17) + roff]
                )
                obuf[sl] = vn + vr
                return carry2

            split = jnp.int32(0)
            for k, (bc, bg) in SPLITS.items():
                split = jnp.where(c == bc, bg, split)
            lax.fori_loop(0, split, row_fn, 0)
            for k, (bc, bg) in SPLITS.items():
                @pl.when(c == bc)
                def _(k=k):
                    stage_table(k)
            ngrp = jnp.where(c == NBLK - 1, TAILG, 256)
            lax.fori_loop(split, ngrp, row_fn, 0)

            @pl.when(c < NBLK - 1)
            def _():
                pltpu.async_copy(
                    obuf, val_hbm.at[pl.ds(wbase + c * 4096, 4096)], sout
                )

            @pl.when(c == NBLK - 1)
            def _():
                pltpu.async_copy(
                    obuf.at[pl.ds(0, TAILW)],
                    val_hbm.at[pl.ds(wbase + c * 4096, TAILW)], sout
                )

            @pl.when(c + 2 < NBLK)
            def _():
                pltpu.async_copy(idx_src(c + 2), ibuf, sin)

        return carry

    lax.fori_loop(0, NBLK // 2, super_fn, 0)
    pltpu.make_async_copy(
        o0, val_hbm.at[pl.ds(wbase + (NBLK - 2) * 4096, 4096)], so0
    ).wait()
    pltpu.make_async_copy(
        o1.at[pl.ds(0, TAILW)],
        val_hbm.at[pl.ds(wbase + (NBLK - 1) * 4096, TAILW)], so1
    ).wait()


def _p2_body(val_ref, out_ref):
    v = val_ref[...]
    bs = v.shape[0] // S
    r = v.reshape(bs, S, D)
    out_ref[...] = jnp.maximum(jnp.sum(r, axis=1) * jnp.float32(1.0 / S), 0.0)


@jax.jit
def kernel(sampled_neighbors, sampled_relations, weight, relation_weight):
    nf = sampled_neighbors.reshape(-1).astype(jnp.int32)
    rf = sampled_relations.reshape(-1).astype(jnp.int32)
    pk = nf | (rf << 17)
    pk = jnp.concatenate([pk, pk, pk, pk])
    pk = jnp.pad(pk, (0, IRP * 128 - WPT))
    side = jnp.concatenate(
        [weight[:, NAL:], relation_weight,
         jnp.zeros((D, 256 - 32 - NREL), jnp.float32)], axis=1)

    mesh = plsc.VectorSubcoreMesh(
        core_axis_name="c", subcore_axis_name="s", num_cores=NC, num_subcores=NS
    )
    val = pl.kernel(
        _p1_body,
        out_type=jax.ShapeDtypeStruct((B * S * D,), jnp.float32),
        mesh=mesh,
        scratch_types=[
            pltpu.VMEM((N + 256 - 32,), jnp.float32),
            pltpu.VMEM((4096,), jnp.int32),
            pltpu.VMEM((4096,), jnp.int32),
            pltpu.VMEM((4096,), jnp.float32),
            pltpu.VMEM((4096,), jnp.float32),
            pltpu.SemaphoreType.DMA,
            pltpu.SemaphoreType.DMA,
            pltpu.SemaphoreType.DMA,
            pltpu.SemaphoreType.DMA,
        ],
        compiler_params=pltpu.CompilerParams(needs_layout_passes=False),
    )(weight, side, pk)

    bs = 2000
    out = pl.pallas_call(
        _p2_body,
        out_shape=jax.ShapeDtypeStruct((B, D), jnp.float32),
        grid=(B // bs,),
        in_specs=[pl.BlockSpec((bs * S, D), lambda i: (i, 0))],
        out_specs=pl.BlockSpec((bs, D), lambda i: (i, 0)),
    )(val.reshape(B * S, D))
    return out


# speedup vs baseline: 2.4035x; 1.4513x over previous
import jax
import jax.numpy as jnp
from jax import lax
from jax.experimental import pallas as pl
from jax.experimental.pallas import tpu as pltpu
from jax.experimental.pallas import tpu_sc as plsc

N = 100000
NREL = 17
B = 10000
S = 10
D = 128
L = B * S

NC = 2
NS = 16
NW = NC * NS
DPW = D // NW
WPT = DPW * L
RPT = WPT // 128
BLK = 32
NFULL = WPT // 4096
TAILG = (WPT - NFULL * 4096) // 16
TAILW = WPT - NFULL * 4096
NBLK = NFULL + 1
IRP = NBLK * BLK
NAL = (N // 128) * 128
SPLITS = {1: (24, 106), 2: (48, 212), 3: (73, 62)}


def _p1_body(w_hbm, side_hbm, pk_hbm, val_hbm, table_v, i0, i1, o0, o1,
             si0, si1, so0, so1):
    wid = lax.axis_index("s") * NC + lax.axis_index("c")
    m17 = jnp.full((16,), 0x1FFFF, jnp.int32)
    roff = jnp.full((16,), N, jnp.int32)
    wbase = wid * WPT

    def stage_table(k):
        dd = wid * DPW + k
        pltpu.sync_copy(w_hbm.at[dd, pl.ds(0, NAL)], table_v.at[pl.ds(0, NAL)])
        pltpu.sync_copy(side_hbm.at[dd], table_v.at[pl.ds(NAL, 256)])

    def idx_src(c):
        return pk_hbm.at[pl.ds(c * 4096, 4096)]

    stage_table(0)
    pltpu.async_copy(idx_src(0), i0, si0)
    pltpu.async_copy(idx_src(1), i1, si1)

    bufs = ((i0, o0, si0, so0), (i1, o1, si1, so1))

    def super_fn(h, carry):
        for par, (ibuf, obuf, sin, sout) in enumerate(bufs):
            c = 2 * h + par
            pltpu.make_async_copy(idx_src(c), ibuf, sin).wait()

            @pl.when(h > 0)
            def _():
                pltpu.make_async_copy(
                    obuf, val_hbm.at[pl.ds(wbase + (c - 2) * 4096, 4096)], sout
                ).wait()

            def gather16(g):
                sl = pl.ds(g * 16, 16)
                pk = ibuf[sl]
                n = lax.bitwise_and(pk, m17)
                vn = plsc.load_gather(table_v, [n])
                vr = plsc.load_gather(
                    table_v, [lax.shift_right_logical(pk, 17) + roff]
                )
                obuf[sl] = vn + vr

            def row_fn(r, carry2):
                for gg in range(8):
                    gather16(r * 8 + gg)
                return carry2

            def grp_fn(g, carry2):
                gather16(g)
                return carry2

            lax.fori_loop(0, BLK, row_fn, 0)
            for k, (bc, bg) in SPLITS.items():
                @pl.when(c == bc)
                def _(k=k):
                    stage_table(k)
            redo = jnp.int32(256)
            for k, (bc, bg) in SPLITS.items():
                redo = jnp.where(c == bc, bg, redo)
            lax.fori_loop(redo, 256, grp_fn, 0)

            @pl.when(c < NBLK - 1)
            def _():
                pltpu.async_copy(
                    obuf, val_hbm.at[pl.ds(wbase + c * 4096, 4096)], sout
                )

            @pl.when(c == NBLK - 1)
            def _():
                pltpu.async_copy(
                    obuf.at[pl.ds(0, TAILW)],
                    val_hbm.at[pl.ds(wbase + c * 4096, TAILW)], sout
                )

            @pl.when(c + 2 < NBLK)
            def _():
                pltpu.async_copy(idx_src(c + 2), ibuf, sin)

        return carry

    lax.fori_loop(0, NBLK // 2, super_fn, 0)
    pltpu.make_async_copy(
        o0, val_hbm.at[pl.ds(wbase + (NBLK - 2) * 4096, 4096)], so0
    ).wait()
    pltpu.make_async_copy(
        o1.at[pl.ds(0, TAILW)],
        val_hbm.at[pl.ds(wbase + (NBLK - 1) * 4096, TAILW)], so1
    ).wait()


def _p2_body(val_ref, out_ref):
    v = val_ref[...]
    bs = v.shape[0] // S
    r = v.reshape(bs, S, D)
    out_ref[...] = jnp.maximum(jnp.sum(r, axis=1) * jnp.float32(1.0 / S), 0.0)


@jax.jit
def kernel(sampled_neighbors, sampled_relations, weight, relation_weight):
    nf = sampled_neighbors.reshape(-1).astype(jnp.int32)
    rf = sampled_relations.reshape(-1).astype(jnp.int32)
    pk = nf | (rf << 17)
    pk = jnp.concatenate([pk, pk, pk, pk])
    pk = jnp.pad(pk, (0, IRP * 128 - WPT))
    side = jnp.concatenate(
        [weight[:, NAL:], relation_weight,
         jnp.zeros((D, 256 - 32 - NREL), jnp.float32)], axis=1)

    mesh = plsc.VectorSubcoreMesh(
        core_axis_name="c", subcore_axis_name="s", num_cores=NC, num_subcores=NS
    )
    val = pl.kernel(
        _p1_body,
        out_type=jax.ShapeDtypeStruct((B * S * D,), jnp.float32),
        mesh=mesh,
        scratch_types=[
            pltpu.VMEM((N + 256 - 32,), jnp.float32),
            pltpu.VMEM((4096,), jnp.int32),
            pltpu.VMEM((4096,), jnp.int32),
            pltpu.VMEM((4096,), jnp.float32),
            pltpu.VMEM((4096,), jnp.float32),
            pltpu.SemaphoreType.DMA,
            pltpu.SemaphoreType.DMA,
            pltpu.SemaphoreType.DMA,
            pltpu.SemaphoreType.DMA,
        ],
        compiler_params=pltpu.CompilerParams(needs_layout_passes=False),
    )(weight, side, pk)

    bs = 2000
    out = pl.pallas_call(
        _p2_body,
        out_shape=jax.ShapeDtypeStruct((B, D), jnp.float32),
        grid=(B // bs,),
        in_specs=[pl.BlockSpec((bs * S, D), lambda i: (i, 0))],
        out_specs=pl.BlockSpec((bs, D), lambda i: (i, 0)),
    )(val.reshape(B * S, D))
    return out


# speedup vs baseline: 2.4048x; 1.0005x over previous
import jax
import jax.numpy as jnp
from jax import lax
from jax.experimental import pallas as pl
from jax.experimental.pallas import tpu as pltpu
from jax.experimental.pallas import tpu_sc as plsc

N = 100000
NREL = 17
B = 10000
S = 10
D = 128
L = B * S

NC = 2
NS = 16
NW = NC * NS
DPW = D // NW
WPT = DPW * L
RPT = WPT // 128
BLK = 32
NFULL = WPT // 4096
TAILG = (WPT - NFULL * 4096) // 16
TAILW = WPT - NFULL * 4096
NBLK = NFULL + 1
IRP = NBLK * BLK
NAL = (N // 128) * 128
SPLITS = {1: (24, 106), 2: (48, 212), 3: (73, 62)}


def _p1_body(w_hbm, side_hbm, pk_hbm, val_hbm, table_v, i0, i1, o0, o1,
             si0, si1, so0, so1):
    wid = lax.axis_index("s") * NC + lax.axis_index("c")
    m17 = jnp.full((16,), 0x1FFFF, jnp.int32)
    roff = jnp.full((16,), N, jnp.int32)
    wbase = wid * WPT

    def stage_table(k):
        dd = wid * DPW + k
        pltpu.sync_copy(w_hbm.at[dd, pl.ds(0, NAL)], table_v.at[pl.ds(0, NAL)])
        pltpu.sync_copy(side_hbm.at[dd], table_v.at[pl.ds(NAL, 256)])

    def idx_src(c):
        return pk_hbm.at[pl.ds(c * 4096, 4096)]

    stage_table(0)
    pltpu.async_copy(idx_src(0), i0, si0)
    pltpu.async_copy(idx_src(1), i1, si1)

    bufs = ((i0, o0, si0, so0), (i1, o1, si1, so1))

    def super_fn(h, carry):
        for par, (ibuf, obuf, sin, sout) in enumerate(bufs):
            c = 2 * h + par
            pltpu.make_async_copy(idx_src(c), ibuf, sin).wait()

            @pl.when(h > 0)
            def _():
                pltpu.make_async_copy(
                    obuf, val_hbm.at[pl.ds(wbase + (c - 2) * 4096, 4096)], sout
                ).wait()

            def gather16(g):
                sl = pl.ds(g * 16, 16)
                pk = ibuf[sl]
                n = lax.bitwise_and(pk, m17)
                vn = plsc.load_gather(table_v, [n])
                vr = plsc.load_gather(
                    table_v, [lax.shift_right_logical(pk, 17) + roff]
                )
                obuf[sl] = vn + vr

            def row_fn(r, carry2):
                for gg in range(16):
                    gather16(r * 16 + gg)
                return carry2

            def grp_fn(g, carry2):
                gather16(g)
                return carry2

            lax.fori_loop(0, BLK // 2, row_fn, 0)
            for k, (bc, bg) in SPLITS.items():
                @pl.when(c == bc)
                def _(k=k):
                    stage_table(k)
            redo = jnp.int32(256)
            for k, (bc, bg) in SPLITS.items():
                redo = jnp.where(c == bc, bg, redo)
            lax.fori_loop(redo, 256, grp_fn, 0)

            @pl.when(c < NBLK - 1)
            def _():
                pltpu.async_copy(
                    obuf, val_hbm.at[pl.ds(wbase + c * 4096, 4096)], sout
                )

            @pl.when(c == NBLK - 1)
            def _():
                pltpu.async_copy(
                    obuf.at[pl.ds(0, TAILW)],
                    val_hbm.at[pl.ds(wbase + c * 4096, TAILW)], sout
                )

            @pl.when(c + 2 < NBLK)
            def _():
                pltpu.async_copy(idx_src(c + 2), ibuf, sin)

        return carry

    lax.fori_loop(0, NBLK // 2, super_fn, 0)
    pltpu.make_async_copy(
        o0, val_hbm.at[pl.ds(wbase + (NBLK - 2) * 4096, 4096)], so0
    ).wait()
    pltpu.make_async_copy(
        o1.at[pl.ds(0, TAILW)],
        val_hbm.at[pl.ds(wbase + (NBLK - 1) * 4096, TAILW)], so1
    ).wait()


def _p2_body(val_ref, out_ref):
    v = val_ref[...]
    bs = v.shape[0] // S
    r = v.reshape(bs, S, D)
    out_ref[...] = jnp.maximum(jnp.sum(r, axis=1) * jnp.float32(1.0 / S), 0.0)


@jax.jit
def kernel(sampled_neighbors, sampled_relations, weight, relation_weight):
    nf = sampled_neighbors.reshape(-1).astype(jnp.int32)
    rf = sampled_relations.reshape(-1).astype(jnp.int32)
    pk = nf | (rf << 17)
    pk = jnp.concatenate([pk, pk, pk, pk])
    pk = jnp.pad(pk, (0, IRP * 128 - WPT))
    side = jnp.concatenate(
        [weight[:, NAL:], relation_weight,
         jnp.zeros((D, 256 - 32 - NREL), jnp.float32)], axis=1)

    mesh = plsc.VectorSubcoreMesh(
        core_axis_name="c", subcore_axis_name="s", num_cores=NC, num_subcores=NS
    )
    val = pl.kernel(
        _p1_body,
        out_type=jax.ShapeDtypeStruct((B * S * D,), jnp.float32),
        mesh=mesh,
        scratch_types=[
            pltpu.VMEM((N + 256 - 32,), jnp.float32),
            pltpu.VMEM((4096,), jnp.int32),
            pltpu.VMEM((4096,), jnp.int32),
            pltpu.VMEM((4096,), jnp.float32),
            pltpu.VMEM((4096,), jnp.float32),
            pltpu.SemaphoreType.DMA,
            pltpu.SemaphoreType.DMA,
            pltpu.SemaphoreType.DMA,
            pltpu.SemaphoreType.DMA,
        ],
        compiler_params=pltpu.CompilerParams(needs_layout_passes=False),
    )(weight, side, pk)

    bs = 2000
    out = pl.pallas_call(
        _p2_body,
        out_shape=jax.ShapeDtypeStruct((B, D), jnp.float32),
        grid=(B // bs,),
        in_specs=[pl.BlockSpec((bs * S, D), lambda i: (i, 0))],
        out_specs=pl.BlockSpec((bs, D), lambda i: (i, 0)),
    )(val.reshape(B * S, D))
    return out
